# Initial kernel scaffold; baseline (speedup 1.0000x reference)
import jax
import jax.numpy as jnp
from jax.experimental import pallas as pl


def kernel(x, embed_weight, proj_weight):
    raise NotImplementedError("write your pallas kernel here")


# speedup vs baseline: 1.2318x; 1.2318x over previous
import functools

import jax
import jax.numpy as jnp
from jax.experimental import pallas as pl
from jax.experimental.pallas import tpu as pltpu
from jax.experimental.pallas import tpu_sc as plsc

INNER = 64
MODEL = 512
_GATHER_W = 128
_MM_T = 2048


def _sc_gather(table, idx):
    n = idx.shape[0]
    idx2 = idx.reshape(1, n)
    mesh = plsc.VectorSubcoreMesh(core_axis_name="c", subcore_axis_name="s")

    @functools.partial(
        pl.kernel,
        out_type=jax.ShapeDtypeStruct((n, INNER), table.dtype),
        mesh=mesh,
        compiler_params=pltpu.CompilerParams(use_tc_tiling_on_sc=False),
    )
    def gather_kernel(table_hbm, idx_hbm, out_hbm):
        def body(i_vmem, o_vmem):
            pltpu.sync_copy(table_hbm.at[i_vmem.at[0]], o_vmem)

        pltpu.emit_pipeline(
            body,
            grid=(n // _GATHER_W,),
            in_specs=[pl.BlockSpec((1, _GATHER_W), lambda i: (0, i))],
            out_specs=[pl.BlockSpec((_GATHER_W, INNER), lambda i: (i, 0))],
            core_axis_name=("c", "s"),
            dimension_semantics=(pltpu.PARALLEL,),
        )(idx_hbm, out_hbm)

    return gather_kernel(table, idx2)


def _mm_body(h_ref, p_ref, o_ref):
    o_ref[...] = jax.lax.dot_general(
        h_ref[...],
        p_ref[...],
        (((1,), (1,)), ((), ())),
        preferred_element_type=jnp.float32,
    )


def _tc_project(h, proj_weight):
    n = h.shape[0]
    return pl.pallas_call(
        _mm_body,
        grid=(n // _MM_T,),
        in_specs=[
            pl.BlockSpec((_MM_T, INNER), lambda i: (i, 0)),
            pl.BlockSpec((MODEL, INNER), lambda i: (0, 0)),
        ],
        out_specs=pl.BlockSpec((_MM_T, MODEL), lambda i: (i, 0)),
        out_shape=jax.ShapeDtypeStruct((n, MODEL), jnp.float32),
    )(h, proj_weight)


def kernel(x, embed_weight, proj_weight):
    b, l = x.shape
    xf = x.reshape(b * l)
    h = _sc_gather(embed_weight, xf)
    out = _tc_project(h, proj_weight)
    return out.reshape(b, l, MODEL)
